# baseline (speedup 1.0000x reference)
import functools

import jax
import jax.numpy as jnp
from jax import lax
from jax.experimental import pallas as pl
from jax.experimental.pallas import tpu as pltpu
from jax.experimental.pallas import tpu_sc as plsc

_NC, _NS, _L = 2, 16, 16
_NW = _NC * _NS


def _sc_gather_pairs(table2, idx):
    _, d2 = table2.shape
    b = idx.shape[0]
    b_per_w = b // _NW
    mesh = plsc.VectorSubcoreMesh(core_axis_name="c", subcore_axis_name="s")

    @functools.partial(
        pl.kernel,
        mesh=mesh,
        out_type=jax.ShapeDtypeStruct((b, d2), jnp.float32),
        scratch_types=[
            pltpu.VMEM((b_per_w,), jnp.int32),
            pltpu.VMEM((b_per_w,), jnp.int32),
            pltpu.VMEM((b_per_w, d2), jnp.float32),
            pltpu.SemaphoreType.DMA,
        ],
    )
    def gather_kernel(table_hbm, idx_hbm, out_hbm, idx_v, idx2_v, rows_v, sem):
        wid = lax.axis_index("s") * _NC + lax.axis_index("c")
        base = wid * b_per_w
        pltpu.sync_copy(idx_hbm.at[pl.ds(base, b_per_w)], idx_v)
        for i in range(b_per_w // _L):
            sl = pl.ds(i * _L, _L)
            idx2_v[sl] = jnp.right_shift(idx_v[sl], 1)
        pltpu.async_copy(table_hbm.at[idx2_v], rows_v, sem).wait()
        pltpu.sync_copy(rows_v, out_hbm.at[pl.ds(base, b_per_w)])

    return gather_kernel(table2, idx)


def _head_matmul(x2, idx, w_head, v_blk=1024):
    t, d2 = x2.shape
    v, d = w_head.shape
    nv = pl.cdiv(v, v_blk)

    def body(x2_ref, idx_ref, w_ref, o_ref, x_ref):
        @pl.when(pl.program_id(0) == 0)
        def _():
            odd = (idx_ref[...] & 1) == 1
            x_ref[...] = jnp.where(odd, x2_ref[:, d:], x2_ref[:, :d])

        o_ref[...] = lax.dot_general(
            x_ref[...], w_ref[...],
            dimension_numbers=(((1,), (1,)), ((), ())),
            preferred_element_type=jnp.float32,
        )

    return pl.pallas_call(
        body,
        grid=(nv,),
        in_specs=[
            pl.BlockSpec((t, d2), lambda j: (0, 0)),
            pl.BlockSpec((t, 1), lambda j: (0, 0)),
            pl.BlockSpec((v_blk, d), lambda j: (j, 0)),
        ],
        out_specs=pl.BlockSpec((t, v_blk), lambda j: (0, j)),
        out_shape=jax.ShapeDtypeStruct((t, v), jnp.float32),
        scratch_shapes=[pltpu.VMEM((t, d), jnp.float32)],
    )(x2, idx, w_head)


def kernel(input_ids, W_emb, W_head):
    b, t = input_ids.shape
    v, d = W_emb.shape
    idx = input_ids.reshape(-1).astype(jnp.int32)
    table2 = W_emb.reshape(v // 2, 2 * d)
    x2 = _sc_gather_pairs(table2, idx)
    logits = _head_matmul(x2, idx.reshape(t, 1), W_head)
    return logits.reshape(b, t, v)


# speedup vs baseline: 1.0280x; 1.0280x over previous
import functools

import jax
import jax.numpy as jnp
from jax import lax
from jax.experimental import pallas as pl
from jax.experimental.pallas import tpu as pltpu
from jax.experimental.pallas import tpu_sc as plsc

_NC, _NS, _L = 2, 16, 16
_NW = _NC * _NS


def _sc_gather(table, idx):
    _, d2 = table.shape
    b = idx.shape[0]
    b_per_w = b // _NW
    mesh = plsc.VectorSubcoreMesh(core_axis_name="c", subcore_axis_name="s")

    @functools.partial(
        pl.kernel,
        mesh=mesh,
        out_type=jax.ShapeDtypeStruct((b, d2), jnp.float32),
        scratch_types=[
            pltpu.VMEM((b_per_w,), jnp.int32),
            pltpu.VMEM((b_per_w, d2), jnp.float32),
            pltpu.SemaphoreType.DMA,
        ],
    )
    def gather_kernel(table_hbm, idx_hbm, out_hbm, idx_v, rows_v, sem):
        wid = lax.axis_index("s") * _NC + lax.axis_index("c")
        base = wid * b_per_w
        pltpu.sync_copy(idx_hbm.at[pl.ds(base, b_per_w)], idx_v)
        pltpu.async_copy(table_hbm.at[idx_v], rows_v, sem).wait()
        pltpu.sync_copy(rows_v, out_hbm.at[pl.ds(base, b_per_w)])

    return gather_kernel(table, idx)


def _head_matmul(x2, w_head, v_blk=1024):
    t, d2 = x2.shape
    v, d = w_head.shape
    nv = pl.cdiv(v, v_blk)

    def body(x2_ref, w_ref, o_ref):
        o_ref[...] = lax.dot_general(
            x2_ref[:, :d], w_ref[...],
            dimension_numbers=(((1,), (1,)), ((), ())),
            preferred_element_type=jnp.float32,
        )

    return pl.pallas_call(
        body,
        grid=(nv,),
        in_specs=[
            pl.BlockSpec((t, d2), lambda j: (0, 0)),
            pl.BlockSpec((v_blk, d), lambda j: (j, 0)),
        ],
        out_specs=pl.BlockSpec((t, v_blk), lambda j: (0, j)),
        out_shape=jax.ShapeDtypeStruct((t, v), jnp.float32),
    )(x2, w_head)


def kernel(input_ids, W_emb, W_head):
    b, t = input_ids.shape
    v, d = W_emb.shape
    idx = input_ids.reshape(-1).astype(jnp.int32)
    table = jnp.pad(W_emb, ((0, 0), (0, d)))
    x2 = _sc_gather(table, idx)
    logits = _head_matmul(x2, W_head)
    return logits.reshape(b, t, v)


# speedup vs baseline: 2.5362x; 2.4672x over previous
import functools

import jax
import jax.numpy as jnp
from jax import lax
from jax.experimental import pallas as pl
from jax.experimental.pallas import tpu as pltpu
from jax.experimental.pallas import tpu_sc as plsc

_NC, _NS, _L = 2, 16, 16
_NW = _NC * _NS


def _sc_gather(table, idx):
    _, d2 = table.shape
    b = idx.shape[0]
    b_per_w = b // _NW
    mesh = plsc.VectorSubcoreMesh(core_axis_name="c", subcore_axis_name="s")

    @functools.partial(
        pl.kernel,
        mesh=mesh,
        out_type=jax.ShapeDtypeStruct((b, d2), jnp.float32),
        scratch_types=[
            pltpu.VMEM((b_per_w,), jnp.int32),
            pltpu.VMEM((b_per_w, d2), jnp.float32),
            pltpu.SemaphoreType.DMA,
        ],
    )
    def gather_kernel(table_hbm, idx_hbm, out_hbm, idx_v, rows_v, sem):
        wid = lax.axis_index("s") * _NC + lax.axis_index("c")
        base = wid * b_per_w
        pltpu.sync_copy(idx_hbm.at[pl.ds(base, b_per_w)], idx_v)
        pltpu.async_copy(table_hbm.at[idx_v], rows_v, sem).wait()
        pltpu.sync_copy(rows_v, out_hbm.at[pl.ds(base, b_per_w)])

    return gather_kernel(table, idx)


def _head_matmul_t(x2, w_head, v_blk=1024):
    t, d2 = x2.shape
    v, d = w_head.shape
    nv = pl.cdiv(v, v_blk)

    def body(x2_ref, w_ref, o_ref):
        o_ref[...] = lax.dot_general(
            w_ref[...], x2_ref[:, :d],
            dimension_numbers=(((1,), (1,)), ((), ())),
            preferred_element_type=jnp.float32,
        )

    return pl.pallas_call(
        body,
        grid=(nv,),
        in_specs=[
            pl.BlockSpec((t, d2), lambda j: (0, 0)),
            pl.BlockSpec((v_blk, d), lambda j: (j, 0)),
        ],
        out_specs=pl.BlockSpec((v_blk, t), lambda j: (j, 0)),
        out_shape=jax.ShapeDtypeStruct((v, t), jnp.float32),
    )(x2, w_head)


def kernel(input_ids, W_emb, W_head):
    b, t = input_ids.shape
    v, d = W_emb.shape
    idx = input_ids.reshape(-1).astype(jnp.int32)
    table = jnp.pad(W_emb, ((0, 0), (0, d)))
    x2 = _sc_gather(table, idx)
    logits_t = _head_matmul_t(x2, W_head)
    return logits_t.T.reshape(b, t, v)


# speedup vs baseline: 2.9041x; 1.1451x over previous
import functools

import jax
import jax.numpy as jnp
from jax import lax
from jax.experimental import pallas as pl
from jax.experimental.pallas import tpu as pltpu
from jax.experimental.pallas import tpu_sc as plsc

_NC, _NS, _L = 2, 16, 16
_NW = _NC * _NS


def _sc_gather(table, idx):
    _, d2 = table.shape
    b = idx.shape[0]
    b_per_w = b // _NW
    mesh = plsc.VectorSubcoreMesh(core_axis_name="c", subcore_axis_name="s")

    @functools.partial(
        pl.kernel,
        mesh=mesh,
        out_type=jax.ShapeDtypeStruct((b, d2), jnp.float32),
        scratch_types=[
            pltpu.VMEM((b_per_w,), jnp.int32),
            pltpu.VMEM((b_per_w, d2), jnp.float32),
            pltpu.SemaphoreType.DMA,
        ],
    )
    def gather_kernel(table_hbm, idx_hbm, out_hbm, idx_v, rows_v, sem):
        wid = lax.axis_index("s") * _NC + lax.axis_index("c")
        base = wid * b_per_w
        pltpu.sync_copy(idx_hbm.at[pl.ds(base, b_per_w)], idx_v)
        pltpu.async_copy(table_hbm.at[idx_v], rows_v, sem).wait()
        pltpu.sync_copy(rows_v, out_hbm.at[pl.ds(base, b_per_w)])

    return gather_kernel(table, idx)


def _head_matmul_t(x2, w_head_t, v_blk=1024):
    t, d2 = x2.shape
    d, v = w_head_t.shape
    nv = pl.cdiv(v, v_blk)

    def body(x2_ref, w_ref, o_ref):
        o_ref[...] = lax.dot_general(
            w_ref[...], x2_ref[:, :d],
            dimension_numbers=(((0,), (1,)), ((), ())),
            preferred_element_type=jnp.float32,
        )

    return pl.pallas_call(
        body,
        grid=(nv,),
        in_specs=[
            pl.BlockSpec((t, d2), lambda j: (0, 0)),
            pl.BlockSpec((d, v_blk), lambda j: (0, j)),
        ],
        out_specs=pl.BlockSpec((v_blk, t), lambda j: (j, 0)),
        out_shape=jax.ShapeDtypeStruct((v, t), jnp.float32),
    )(x2, w_head_t)


def kernel(input_ids, W_emb, W_head):
    b, t = input_ids.shape
    v, d = W_emb.shape
    idx = input_ids.reshape(-1).astype(jnp.int32)
    table = jnp.pad(W_emb, ((0, 0), (0, d)))
    x2 = _sc_gather(table, idx)
    logits_t = _head_matmul_t(x2, W_head.T)
    return logits_t.T.reshape(b, t, v)


# speedup vs baseline: 2.9466x; 1.0146x over previous
import functools

import jax
import jax.numpy as jnp
from jax import lax
from jax.experimental import pallas as pl
from jax.experimental.pallas import tpu as pltpu
from jax.experimental.pallas import tpu_sc as plsc

_NC, _NS, _L = 2, 16, 16
_NW = _NC * _NS


def _sc_gather(table, idx):
    _, d2 = table.shape
    b = idx.shape[0]
    b_per_w = b // _NW
    mesh = plsc.VectorSubcoreMesh(core_axis_name="c", subcore_axis_name="s")

    @functools.partial(
        pl.kernel,
        mesh=mesh,
        out_type=jax.ShapeDtypeStruct((b, d2), jnp.float32),
        scratch_types=[
            pltpu.VMEM((b_per_w,), jnp.int32),
            pltpu.VMEM((b_per_w, d2), jnp.float32),
            pltpu.SemaphoreType.DMA,
        ],
    )
    def gather_kernel(table_hbm, idx_hbm, out_hbm, idx_v, rows_v, sem):
        wid = lax.axis_index("s") * _NC + lax.axis_index("c")
        base = wid * b_per_w
        pltpu.sync_copy(idx_hbm.at[pl.ds(base, b_per_w)], idx_v)
        pltpu.async_copy(table_hbm.at[idx_v], rows_v, sem).wait()
        pltpu.sync_copy(rows_v, out_hbm.at[pl.ds(base, b_per_w)])

    return gather_kernel(table, idx)


def _pad_table(w_emb_t, t_blk=2048):
    d, v = w_emb_t.shape
    nb = pl.cdiv(v, t_blk)

    def body(w_ref, o_ref):
        o_ref[:, :d] = w_ref[...].T
        o_ref[:, d:] = jnp.zeros_like(o_ref[:, d:])

    return pl.pallas_call(
        body,
        grid=(nb,),
        in_specs=[pl.BlockSpec((d, t_blk), lambda j: (0, j))],
        out_specs=pl.BlockSpec((t_blk, 2 * d), lambda j: (j, 0)),
        out_shape=jax.ShapeDtypeStruct((v, 2 * d), jnp.float32),
    )(w_emb_t)


def _head_matmul_t(x2, w_head_t, v_blk=1024):
    t, d2 = x2.shape
    d, v = w_head_t.shape
    nv = pl.cdiv(v, v_blk)

    def body(x2_ref, w_ref, o_ref):
        o_ref[...] = lax.dot_general(
            w_ref[...], x2_ref[:, :d],
            dimension_numbers=(((0,), (1,)), ((), ())),
            preferred_element_type=jnp.float32,
        )

    return pl.pallas_call(
        body,
        grid=(nv,),
        in_specs=[
            pl.BlockSpec((t, d2), lambda j: (0, 0)),
            pl.BlockSpec((d, v_blk), lambda j: (0, j)),
        ],
        out_specs=pl.BlockSpec((v_blk, t), lambda j: (j, 0)),
        out_shape=jax.ShapeDtypeStruct((v, t), jnp.float32),
    )(x2, w_head_t)


def kernel(input_ids, W_emb, W_head):
    b, t = input_ids.shape
    v, d = W_emb.shape
    idx = input_ids.reshape(-1).astype(jnp.int32)
    table = _pad_table(W_emb.T)
    x2 = _sc_gather(table, idx)
    logits_t = _head_matmul_t(x2, W_head.T)
    return logits_t.T.reshape(b, t, v)


# speedup vs baseline: 3.0102x; 1.0216x over previous
import functools

import jax
import jax.numpy as jnp
from jax import lax
from jax.experimental import pallas as pl
from jax.experimental.pallas import tpu as pltpu
from jax.experimental.pallas import tpu_sc as plsc

_NC, _NS, _L = 2, 16, 16
_NW = _NC * _NS


def _sc_gather_dmajor(w_flat, idx, d, v):
    b = idx.shape[0]
    tb = 128
    n_tok_blocks = b // tb
    n_feat_splits = _NW // n_tok_blocks
    fb = d // n_feat_splits
    mesh = plsc.VectorSubcoreMesh(core_axis_name="c", subcore_axis_name="s")

    @functools.partial(
        pl.kernel,
        mesh=mesh,
        out_type=jax.ShapeDtypeStruct((d, b), jnp.float32),
        scratch_types=[
            pltpu.VMEM((tb,), jnp.int32),
            pltpu.VMEM((fb, tb), jnp.int32),
            pltpu.VMEM((fb, tb), jnp.float32),
            pltpu.SemaphoreType.DMA,
        ],
    )
    def gather_kernel(w_hbm, idx_hbm, out_hbm, idx_v, fidx_v, rows_v, sem):
        wid = lax.axis_index("s") * _NC + lax.axis_index("c")
        k = wid % n_tok_blocks
        h = wid // n_tok_blocks
        tok0 = k * tb
        f0 = h * fb
        pltpu.sync_copy(idx_hbm.at[pl.ds(tok0, tb)], idx_v)

        def fill(f, c):
            for i in range(tb // _L):
                sl = pl.ds(i * _L, _L)
                fidx_v[f, sl] = idx_v[sl] + (f0 + f) * v
            return c

        lax.fori_loop(0, fb, fill, 0)

        def fire(f, c):
            pltpu.async_copy(w_hbm.at[fidx_v.at[f]], rows_v.at[f], sem)
            return c

        lax.fori_loop(0, fb, fire, 0)

        def drain(f, c):
            pltpu.make_async_copy(w_hbm.at[pl.ds(0, tb)], rows_v.at[f], sem).wait()
            return c

        lax.fori_loop(0, fb, drain, 0)
        pltpu.sync_copy(rows_v, out_hbm.at[pl.ds(f0, fb), pl.ds(tok0, tb)])

    return gather_kernel(w_flat, idx)


def _head_matmul_t(x_t, w_head_t, v_blk=1024):
    d, t = x_t.shape
    _, v = w_head_t.shape
    nv = pl.cdiv(v, v_blk)

    def body(x_ref, w_ref, o_ref):
        o_ref[...] = lax.dot_general(
            w_ref[...], x_ref[...],
            dimension_numbers=(((0,), (0,)), ((), ())),
            preferred_element_type=jnp.float32,
        )

    return pl.pallas_call(
        body,
        grid=(nv,),
        in_specs=[
            pl.BlockSpec((d, t), lambda j: (0, 0)),
            pl.BlockSpec((d, v_blk), lambda j: (0, j)),
        ],
        out_specs=pl.BlockSpec((v_blk, t), lambda j: (j, 0)),
        out_shape=jax.ShapeDtypeStruct((v, t), jnp.float32),
    )(x_t, w_head_t)


def kernel(input_ids, W_emb, W_head):
    b, t = input_ids.shape
    v, d = W_emb.shape
    idx = input_ids.reshape(-1).astype(jnp.int32)
    w_flat = W_emb.T.reshape(-1)
    x_t = _sc_gather_dmajor(w_flat, idx, d, v)
    logits_t = _head_matmul_t(x_t, W_head.T)
    return logits_t.T.reshape(b, t, v)


# speedup vs baseline: 3.0403x; 1.0100x over previous
import functools

import jax
import jax.numpy as jnp
from jax import lax
from jax.experimental import pallas as pl
from jax.experimental.pallas import tpu as pltpu
from jax.experimental.pallas import tpu_sc as plsc

_NC, _NS, _L = 2, 16, 16
_NW = _NC * _NS


def _sc_gather_dmajor(w_flat, idx, d, v):
    b = idx.shape[0]
    tb = 128
    n_tok_blocks = b // tb
    n_feat_splits = _NW // n_tok_blocks
    fb = d // n_feat_splits
    mesh = plsc.VectorSubcoreMesh(core_axis_name="c", subcore_axis_name="s")

    @functools.partial(
        pl.kernel,
        mesh=mesh,
        out_type=jax.ShapeDtypeStruct((d, b), jnp.float32),
        scratch_types=[
            pltpu.VMEM((tb,), jnp.int32),
            pltpu.VMEM((fb, tb), jnp.int32),
            pltpu.VMEM((fb, tb), jnp.float32),
            pltpu.SemaphoreType.DMA,
        ],
    )
    def gather_kernel(w_hbm, idx_hbm, out_hbm, idx_v, fidx_v, rows_v, sem):
        wid = lax.axis_index("s") * _NC + lax.axis_index("c")
        k = wid % n_tok_blocks
        h = wid // n_tok_blocks
        tok0 = k * tb
        f0 = h * fb
        pltpu.sync_copy(idx_hbm.at[pl.ds(tok0, tb)], idx_v)

        def fill(f, c):
            for i in range(tb // _L):
                sl = pl.ds(i * _L, _L)
                fidx_v[f, sl] = idx_v[sl] + (f0 + f) * v
            return c

        lax.fori_loop(0, fb, fill, 0)

        def fire(f, c):
            pltpu.async_copy(w_hbm.at[fidx_v.at[f]], rows_v.at[f], sem)
            return c

        lax.fori_loop(0, fb, fire, 0)

        def drain(f, c):
            pltpu.make_async_copy(w_hbm.at[pl.ds(0, tb)], rows_v.at[f], sem).wait()
            return c

        lax.fori_loop(0, fb, drain, 0)
        pltpu.sync_copy(rows_v, out_hbm.at[pl.ds(f0, fb), pl.ds(tok0, tb)])

    return gather_kernel(w_flat, idx)


def _head_matmul_t(x_t, w_head_t, v_blk=2048):
    d, t = x_t.shape
    _, v = w_head_t.shape
    nv = pl.cdiv(v, v_blk)

    def body(x_ref, w_ref, o_ref):
        o_ref[...] = lax.dot_general(
            w_ref[...], x_ref[...],
            dimension_numbers=(((0,), (0,)), ((), ())),
            preferred_element_type=jnp.float32,
        )

    return pl.pallas_call(
        body,
        grid=(nv,),
        in_specs=[
            pl.BlockSpec((d, t), lambda j: (0, 0)),
            pl.BlockSpec((d, v_blk), lambda j: (0, j)),
        ],
        out_specs=pl.BlockSpec((v_blk, t), lambda j: (j, 0)),
        out_shape=jax.ShapeDtypeStruct((v, t), jnp.float32),
    )(x_t, w_head_t)


def kernel(input_ids, W_emb, W_head):
    b, t = input_ids.shape
    v, d = W_emb.shape
    idx = input_ids.reshape(-1).astype(jnp.int32)
    w_flat = W_emb.T.reshape(-1)
    x_t = _sc_gather_dmajor(w_flat, idx, d, v)
    logits_t = _head_matmul_t(x_t, W_head.T)
    return logits_t.T.reshape(b, t, v)
